# Initial kernel scaffold; baseline (speedup 1.0000x reference)
import jax
import jax.numpy as jnp
from jax.experimental import pallas as pl


def kernel(x_note, edge_index, batch, W_embed, b_embed, Ws1, Wn1, bS1, Ws2, Wn2, bS2, W1s, W1n, b1, W2s, W2n, b2):
    raise NotImplementedError("write your pallas kernel here")


# speedup vs baseline: 14.4704x; 14.4704x over previous
import functools

import jax
import jax.numpy as jnp
from jax import lax
from jax.experimental import pallas as pl
from jax.experimental.pallas import tpu as pltpu
from jax.experimental.pallas import tpu_sc as plsc

N = 10000
D = 32
C = 15
CP = 16
H = 200
NC = 2
NS = 16
NW = NC * NS
CH = 128
ACC_ROWS = N + 16
NEG = -1.0e30


def _sc_pass_body(with_ones, n_chunks, *refs):
    if with_ones:
        (table, src, dst, zacc, zdeg, ones_hbm, acc_out, deg_out,
         idx_s, idx_d, rows, sem, ones_v, acc_sh, deg_sh) = refs
    else:
        (table, src, dst, zacc, acc_out,
         idx_s, idx_d, rows, sem, acc_sh) = refs
    cid = lax.axis_index("c")
    sid = lax.axis_index("s")
    wid = sid * NC + cid

    @pl.when(sid == 0)
    def _():
        pltpu.sync_copy(zacc, acc_sh)
        if with_ones:
            pltpu.sync_copy(zdeg, deg_sh)

    if with_ones:
        pltpu.sync_copy(ones_hbm, ones_v)
    plsc.subcore_barrier()

    base0 = wid * (n_chunks * CH)

    @pl.loop(0, n_chunks)
    def _(j):
        base = pl.multiple_of(base0 + j * CH, 8)
        pltpu.sync_copy(src.at[pl.ds(base, CH)], idx_s)
        pltpu.sync_copy(dst.at[pl.ds(base, CH)], idx_d)
        pltpu.async_copy(table.at[idx_s], rows, sem).wait()
        pltpu.sync_copy(rows, acc_sh.at[idx_d], add=True)
        if with_ones:
            pltpu.sync_copy(ones_v, deg_sh.at[idx_d], add=True)

    plsc.subcore_barrier()

    @pl.when(sid == 0)
    def _():
        pltpu.sync_copy(acc_sh, acc_out.at[cid])
        if with_ones:
            pltpu.sync_copy(deg_sh, deg_out.at[cid])


@functools.lru_cache(maxsize=None)
def _make_sc_pass(width, n_chunks, with_ones):
    mesh = plsc.VectorSubcoreMesh(core_axis_name="c", subcore_axis_name="s")
    out_type = [jax.ShapeDtypeStruct((NC, ACC_ROWS, width), jnp.float32)]
    scratch = [
        pltpu.VMEM((CH,), jnp.int32),
        pltpu.VMEM((CH,), jnp.int32),
        pltpu.VMEM((CH, width), jnp.float32),
        pltpu.SemaphoreType.DMA,
    ]
    if with_ones:
        out_type.append(jax.ShapeDtypeStruct((NC, ACC_ROWS, CP), jnp.float32))
        scratch.append(pltpu.VMEM((CH, CP), jnp.float32))
    scratch.append(pltpu.VMEM_SHARED((ACC_ROWS, width), jnp.float32))
    if with_ones:
        scratch.append(pltpu.VMEM_SHARED((ACC_ROWS, CP), jnp.float32))
    return pl.kernel(
        functools.partial(_sc_pass_body, with_ones, n_chunks),
        out_type=tuple(out_type) if len(out_type) > 1 else out_type[0],
        mesh=mesh,
        scratch_types=scratch,
        compiler_params=pltpu.CompilerParams(use_tc_tiling_on_sc=False),
    )


def _embed_body(xn_ref, w_ref, b_ref, o_ref):
    o_ref[...] = (
        jnp.dot(xn_ref[...], w_ref[...], preferred_element_type=jnp.float32)
        + b_ref[...]
    )


def _softmax(logits):
    m = jnp.max(logits, axis=-1, keepdims=True)
    e = jnp.exp(logits - m)
    return e / jnp.sum(e, axis=-1, keepdims=True)


def _mid_body(x_ref, aggp_ref, degp_ref, ws_ref, wn_ref, b_ref,
              s1p_ref, x1p_ref):
    x = x_ref[...]
    aggsum = aggp_ref[0, :N, :] + aggp_ref[1, :N, :]
    deg = jnp.maximum(degp_ref[0, :N, 0:1] + degp_ref[1, :N, 0:1], 1.0)
    agg = aggsum / deg
    logits = (
        jnp.dot(x, ws_ref[...], preferred_element_type=jnp.float32)
        + jnp.dot(agg, wn_ref[...], preferred_element_type=jnp.float32)
        + b_ref[...]
    )
    s1p = _softmax(logits)
    s1p_ref[...] = s1p
    x1p_ref[...] = lax.dot_general(
        s1p, x, (((0,), (0,)), ((), ())), preferred_element_type=jnp.float32)


def _final_body(tp_ref, s1p_ref, x1p_ref, ws2_ref, wn2_ref, bs2_ref,
                w1s_ref, w1n_ref, b1_ref, w2s_ref, w2n_ref, b2_ref,
                z2_ref, s2_ref):
    f32 = jnp.float32
    T = tp_ref[0, :N, :] + tp_ref[1, :N, :]
    s1p = s1p_ref[...]
    A1 = lax.dot_general(T, s1p, (((0,), (0,)), ((), ())),
                         preferred_element_type=f32)
    x1 = x1p_ref[...]
    d1 = jnp.maximum(jnp.sum(A1, axis=1, keepdims=True), 1e-6)
    agg1 = jnp.dot(A1, x1, preferred_element_type=f32) / d1
    S2 = _softmax(
        jnp.dot(x1, ws2_ref[...], preferred_element_type=f32)
        + jnp.dot(agg1, wn2_ref[...], preferred_element_type=f32)
        + bs2_ref[...])
    x2 = lax.dot_general(S2, x1, (((0,), (0,)), ((), ())),
                         preferred_element_type=f32)
    A2 = lax.dot_general(S2, jnp.dot(A1, S2, preferred_element_type=f32),
                         (((0,), (0,)), ((), ())), preferred_element_type=f32)
    d2 = jnp.maximum(jnp.sum(A2, axis=1, keepdims=True), 1e-6)
    agg2 = jnp.dot(A2, x2, preferred_element_type=f32) / d2
    h = jnp.maximum(
        jnp.dot(x2, w1s_ref[...], preferred_element_type=f32)
        + jnp.dot(agg2, w1n_ref[...], preferred_element_type=f32)
        + b1_ref[...], 0.0)
    aggh = jnp.dot(A2, h, preferred_element_type=f32) / d2
    z2_ref[...] = (
        jnp.dot(h, w2s_ref[...], preferred_element_type=f32)
        + jnp.dot(aggh, w2n_ref[...], preferred_element_type=f32)
        + b2_ref[...])
    s2_ref[...] = S2


def _pad_w(w):
    return jnp.concatenate([w, jnp.zeros((w.shape[0], CP - C), jnp.float32)], axis=1)


def _pad_b(b):
    return jnp.concatenate([b, jnp.full((CP - C,), NEG, jnp.float32)]).reshape(1, CP)


def kernel(x_note, edge_index, batch, W_embed, b_embed, Ws1, Wn1, bS1,
           Ws2, Wn2, bS2, W1s, W1n, b1, W2s, W2n, b2):
    E = edge_index.shape[1]
    quant = NW * CH
    E_pad = ((E + quant - 1) // quant) * quant
    n_chunks = E_pad // (NW * CH)
    pad = E_pad - E
    src = jnp.concatenate([edge_index[0], jnp.zeros((pad,), jnp.int32)])
    dst = jnp.concatenate([edge_index[1], jnp.full((pad,), N, jnp.int32)])

    x = pl.pallas_call(
        _embed_body,
        out_shape=jax.ShapeDtypeStruct((N, D), jnp.float32),
    )(x_note, W_embed, b_embed.reshape(1, D))

    zacc = jnp.zeros((ACC_ROWS, D), jnp.float32)
    zdeg = jnp.zeros((ACC_ROWS, CP), jnp.float32)
    ones = jnp.ones((CH, CP), jnp.float32)
    aggp, degp = _make_sc_pass(D, n_chunks, True)(
        x, src, dst, zacc, zdeg, ones)

    s1p, x1p = pl.pallas_call(
        _mid_body,
        out_shape=(
            jax.ShapeDtypeStruct((N, CP), jnp.float32),
            jax.ShapeDtypeStruct((CP, D), jnp.float32),
        ),
    )(x, aggp, degp, _pad_w(Ws1), _pad_w(Wn1), _pad_b(bS1))

    zacc2 = jnp.zeros((ACC_ROWS, CP), jnp.float32)
    tp = _make_sc_pass(CP, n_chunks, False)(s1p, src, dst, zacc2)

    z2p, s2p = pl.pallas_call(
        _final_body,
        out_shape=(
            jax.ShapeDtypeStruct((CP, H), jnp.float32),
            jax.ShapeDtypeStruct((CP, CP), jnp.float32),
        ),
    )(tp, s1p, x1p, _pad_w(Ws2), _pad_w(Wn2), _pad_b(bS2),
      W1s, W1n, b1.reshape(1, H), W2s, W2n, b2.reshape(1, H))

    return (z2p[:C], s1p[:, :C], s2p[:C, :C])
